# baseline (speedup 1.0000x reference)
import jax
import jax.numpy as jnp
from jax import lax
from jax.experimental import pallas as pl
from jax.experimental.pallas import tpu as pltpu
from jax.experimental.pallas import tpu_sc as plsc

DIM_ = 512
D2 = 1024
KC = 128
NB = 2048

NWORK = 32
BPW = NB // NWORK


def _sc_gather_body(table_hbm, idx_hbm, out_hbm, idx_v, rows_v, sem):
    wid = lax.axis_index("s") * 2 + lax.axis_index("c")
    base = wid * BPW
    pltpu.sync_copy(idx_hbm.at[pl.ds(base, BPW)], idx_v)
    pltpu.async_copy(table_hbm.at[idx_v], rows_v, sem).wait()
    pltpu.sync_copy(rows_v, out_hbm.at[pl.ds(base, BPW)])


def _sc_gather(table, idx):
    mesh = plsc.VectorSubcoreMesh(core_axis_name="c", subcore_axis_name="s")
    k = pl.kernel(
        _sc_gather_body,
        mesh=mesh,
        out_type=jax.ShapeDtypeStruct((NB, D2), jnp.float32),
        scratch_types=[
            pltpu.VMEM((BPW,), jnp.int32),
            pltpu.VMEM((BPW, D2), jnp.float32),
            pltpu.SemaphoreType.DMA,
        ],
    )
    return k(table, idx)


BM = 256


def _dot(a, b):
    return jnp.dot(a, b, preferred_element_type=jnp.float32)


def _dot_t(a, b):
    return lax.dot_general(a, b, (((1,), (1,)), ((), ())),
                           preferred_element_type=jnp.float32)


def _stage_rows(gate_ref, emb_ref,
                qwr, qwi, kwr, kwi, vwr, vwi,
                bq_ref, bk_ref, bv_ref,
                ngam_ref, nbet_ref, vqe_ref,
                swr, swi, bs_ref, visp_ref, audp_ref,
                cwr, cwi, bc_ref,
                f_ref, row0):
    scale = 1.0 - jax.nn.sigmoid(gate_ref[0, 0])
    gw = emb_ref[pl.ds(row0, BM), :] * scale
    xr = gw[:, :DIM_].astype(jnp.bfloat16)
    xi = gw[:, DIM_:].astype(jnp.bfloat16)

    def _clin(ar, ai, wr, wi, bias):
        out_r = _dot(ar, wr[...]) - _dot(ai, wi[...]) + bias[:, :DIM_]
        out_i = _dot(ar, wi[...]) + _dot(ai, wr[...]) + bias[:, DIM_:]
        return out_r, out_i

    q_r, q_i = _clin(xr, xi, qwr, qwi, bq_ref[...])
    k_r, k_i = _clin(xr, xi, kwr, kwi, bk_ref[...])
    v_r, v_i = _clin(xr, xi, vwr, vwi, bv_ref[...])
    score = jnp.sum(q_r * k_r + q_i * k_i, axis=-1, keepdims=True)
    gate = jax.nn.sigmoid(score)

    def _ln(x):
        m = jnp.mean(x, axis=-1, keepdims=True)
        var = jnp.mean((x - m) ** 2, axis=-1, keepdims=True)
        return (x - m) * lax.rsqrt(var + 1e-5)

    zr = _ln(v_r * gate)
    zi = _ln(v_i * gate)
    zf = jnp.concatenate([zr, zi], axis=-1)
    zf = zf * ngam_ref[...] + nbet_ref[...]

    vqe = vqe_ref[...]
    t = _dot_t(zf.astype(jnp.bfloat16), vqe.astype(jnp.bfloat16))
    ones = jnp.ones((1, D2), jnp.float32)
    ysq = _dot_t(ones, vqe * vqe)
    d = ysq - 2.0 * t
    dmin = jnp.min(d, axis=-1, keepdims=True)
    iot = lax.broadcasted_iota(jnp.int32, (BM, KC), 1)
    am = jnp.min(jnp.where(d <= dmin, iot, KC), axis=-1, keepdims=True)
    oh = (iot == am).astype(jnp.bfloat16)
    zq = _dot(oh, vqe.astype(jnp.bfloat16))

    zqr = zq[:, :DIM_].astype(jnp.bfloat16)
    zqi = zq[:, DIM_:].astype(jnp.bfloat16)
    s_r, s_i = _clin(zqr, zqi, swr, swi, bs_ref[...])
    sflat = jnp.concatenate([s_r, s_i], axis=-1).astype(jnp.bfloat16)

    def _palette(pal_ref):
        pal = pal_ref[...].astype(jnp.bfloat16)
        logit = _dot_t(sflat, pal)
        logit = logit - jnp.max(logit, axis=-1, keepdims=True)
        e = jnp.exp(logit)
        attn = e / jnp.sum(e, axis=-1, keepdims=True)
        return _dot(attn.astype(jnp.bfloat16), pal)

    vo = _palette(visp_ref)
    ao = _palette(audp_ref)

    c_r, c_i = _clin(zf[:, :DIM_].astype(jnp.bfloat16),
                     zf[:, DIM_:].astype(jnp.bfloat16),
                     cwr, cwi, bc_ref[...])

    fr = zf[:, :DIM_] + (vo[:, :DIM_] - ao[:, DIM_:]) - c_i
    fi = zf[:, DIM_:] + (vo[:, DIM_:] + ao[:, :DIM_]) + c_r
    f_ref[pl.ds(row0, BM), :] = (
        jnp.concatenate([fr, fi], axis=-1).astype(jnp.bfloat16))


BND = 1024
NRING = 3


def _fused_body(gate_ref, emb_ref,
                qwr, qwi, kwr, kwi, vwr, vwi,
                bq_ref, bk_ref, bv_ref,
                ngam_ref, nbet_ref, vqe_ref,
                swr, swi, bs_ref, visp_ref, audp_ref,
                cwr, cwi, bc_ref,
                dw_hbm, db_ref,
                out_ref, f_ref, dw_ring, dw_sem):
    j = pl.program_id(0)
    nj = pl.num_programs(0)

    def _dma(slot, blk):
        return pltpu.make_async_copy(
            dw_hbm.at[:, pl.ds(blk * BND, BND)],
            dw_ring.at[slot], dw_sem.at[slot])

    @pl.when(j == 0)
    def _stage_a():
        for s in range(NRING):
            _dma(s, s).start()
        for i in range(NB // BM):
            _stage_rows(gate_ref, emb_ref, qwr, qwi, kwr, kwi, vwr, vwi,
                        bq_ref, bk_ref, bv_ref, ngam_ref, nbet_ref, vqe_ref,
                        swr, swi, bs_ref, visp_ref, audp_ref, cwr, cwi,
                        bc_ref, f_ref, i * BM)

    slot = lax.rem(j, NRING)
    _dma(slot, j).wait()
    dwb = dw_ring[slot].astype(jnp.bfloat16)
    out_ref[...] = jnp.dot(f_ref[...], dwb,
                           preferred_element_type=jnp.float32) + db_ref[...]
    nxt = j + NRING

    @pl.when(nxt < nj)
    def _refill():
        _dma(slot, nxt).start()


def _fused_call(gate2d, emb, qwr, qwi, kwr, kwi, vwr, vwi, bq, bk, bv,
                ngam, nbet, vqe, swr, swi, bs, visp, audp, cwr, cwi, bc,
                dw, db2d, interpret=False):
    nj = 8192 // BND
    const = lambda shape: pl.BlockSpec(shape, lambda j: (0, 0))
    w = const((DIM_, DIM_))
    b = const((1, D2))
    return pl.pallas_call(
        _fused_body,
        grid=(nj,),
        in_specs=[
            pl.BlockSpec((1, 1), lambda j: (0, 0), memory_space=pltpu.SMEM),
            const((NB, D2)),
            w, w, w, w, w, w,
            b, b, b,
            b, b,
            const((KC, D2)),
            w, w, b,
            const((32, D2)),
            const((32, D2)),
            w, w, b,
            pl.BlockSpec(memory_space=pl.ANY),
            pl.BlockSpec((1, BND), lambda j: (0, j)),
        ],
        out_specs=pl.BlockSpec((NB, BND), lambda j: (0, j)),
        out_shape=jax.ShapeDtypeStruct((NB, 8192), jnp.float32),
        scratch_shapes=[pltpu.VMEM((NB, D2), jnp.bfloat16),
                        pltpu.VMEM((NRING, D2, BND), jnp.float32),
                        pltpu.SemaphoreType.DMA((NRING,))],
        compiler_params=pltpu.CompilerParams(
            dimension_semantics=("arbitrary",)),
        interpret=interpret,
    )(gate2d, emb, qwr, qwi, kwr, kwi, vwr, vwi, bq, bk, bv,
      ngam, nbet, vqe, swr, swi, bs, visp, audp, cwr, cwi, bc, dw, db2d)


def _bias2(br, bi):
    return jnp.concatenate([br - bi, br + bi])[None, :]


def kernel(tokens, enc_table, input_gate,
           qWr, qbr, qWi, qbi,
           kWr, kbr, kWi, kbi,
           vWr, vbr, vWi, vbi,
           sWr, sbr, sWi, sbi,
           cWr, cbr, cWi, cbi,
           mgW, mgb, maW, mab,
           mn_gr, mn_br, mn_gi, mn_bi,
           n_gr, n_br, n_gi, n_bi,
           vq_E, vis_P, aud_P, dW, db):
    emb = _sc_gather(enc_table, tokens.astype(jnp.int32))

    b16 = lambda w: w.astype(jnp.bfloat16)
    ngam = jnp.concatenate([n_gr, n_gi])[None, :]
    nbet = jnp.concatenate([n_br, n_bi])[None, :]
    gate2d = jnp.reshape(input_gate, (1, 1)).astype(jnp.float32)

    logits = _fused_call(gate2d, emb,
                         b16(qWr), b16(qWi), b16(kWr), b16(kWi), b16(vWr),
                         b16(vWi), _bias2(qbr, qbi), _bias2(kbr, kbi),
                         _bias2(vbr, vbi), ngam, nbet, vq_E,
                         b16(sWr), b16(sWi), _bias2(sbr, sbi), vis_P, aud_P,
                         b16(cWr), b16(cWi), _bias2(cbr, cbi),
                         dW, db[None, :])
    return logits
